# baseline (speedup 1.0000x reference)
import functools

import jax
import jax.numpy as jnp
from jax import lax
from jax.experimental import pallas as pl
from jax.experimental.pallas import tpu as pltpu
from jax.experimental.pallas import tpu_sc as plsc

_NUM_EMB = 1000000
_HALF = 32
_BATCH = 16384
_FIELDS = 26
_BF = _BATCH * _FIELDS
_NC = 2
_NS = 16
_NW = _NC * _NS
_PER_W = _BF // _NW
_GRP = 128
_NG = _PER_W // _GRP
_BLK = 512
_GPB = _BLK // _GRP
_NBLK = _PER_W // _BLK

_VT_FULL = _NUM_EMB // _GRP
_VT_ROUNDS = -(-_VT_FULL // _NW)
_TAIL = _NUM_EMB - _VT_FULL * _GRP
_OROWS = _NUM_EMB * _HALF // 128


def _relayout(t1t, t2t, tail1, tail2):
    mesh = plsc.VectorSubcoreMesh(core_axis_name="c", subcore_axis_name="s")
    otype = jax.ShapeDtypeStruct((_OROWS, 128), jnp.float32)

    @functools.partial(
        pl.kernel,
        mesh=mesh,
        compiler_params=pltpu.CompilerParams(needs_layout_passes=False),
        out_type=(otype, otype),
        scratch_types=[
            pltpu.VMEM((_HALF, _GRP), jnp.float32),
            pltpu.VMEM((_HALF, _GRP), jnp.float32),
        ],
    )
    def k(t1_hbm, t2_hbm, tail1_hbm, tail2_hbm, o1_hbm, o2_hbm, sv, dv):
        wid = lax.axis_index("s") * _NC + lax.axis_index("c")
        iota = lax.iota(jnp.int32, 16)

        def round_body(kk, carry):
            t = kk * _NW + wid

            @pl.when(t < _VT_FULL)
            def _():
                for src, dst in ((t1_hbm, o1_hbm), (t2_hbm, o2_hbm)):
                    pltpu.sync_copy(src.at[:, pl.ds(t * _GRP, _GRP)], sv)
                    for r in range(_HALF):
                        for c0 in range(0, 128, 16):
                            rows = iota + (c0 % _HALF)
                            cols = jnp.full((16,), 4 * r + c0 // _HALF,
                                            jnp.int32)
                            dv[r, pl.ds(c0, 16)] = plsc.load_gather(
                                sv, [rows, cols])
                    pltpu.sync_copy(dv, dst.at[pl.ds(t * _HALF, _HALF), :])
            return carry

        lax.fori_loop(0, _VT_ROUNDS, round_body, 0)

        @pl.when(wid == 0)
        def _():
            for tl, dst in ((tail1_hbm, o1_hbm), (tail2_hbm, o2_hbm)):
                pltpu.sync_copy(tl, dv.at[pl.ds(0, _TAIL // 4)])
                pltpu.sync_copy(
                    dv.at[pl.ds(0, _TAIL // 4)],
                    dst.at[pl.ds(_VT_FULL * _HALF, _TAIL // 4), :])

    return k(t1t, t2t, tail1, tail2)


def _dual_gather(x_grp, table1, table2):
    mesh = plsc.VectorSubcoreMesh(core_axis_name="c", subcore_axis_name="s")

    @functools.partial(
        pl.kernel,
        mesh=mesh,
        compiler_params=pltpu.CompilerParams(use_tc_tiling_on_sc=False),
        out_type=jax.ShapeDtypeStruct((_BF, 2 * _HALF), jnp.float32),
        scratch_types=[
            pltpu.VMEM((_NG, _GRP), jnp.int32),
            pltpu.VMEM((2, _BLK, _HALF), jnp.float32),
            pltpu.VMEM((2, _BLK, _HALF), jnp.float32),
            pltpu.SemaphoreType.DMA,
            pltpu.SemaphoreType.DMA,
        ],
    )
    def k(x_hbm, t1_hbm, t2_hbm, out_hbm, idx_v, rows1_v, rows2_v,
          gsem, wsem):
        wid = lax.axis_index("s") * _NC + lax.axis_index("c")
        base = wid * _PER_W
        pltpu.sync_copy(x_hbm.at[wid], idx_v)

        def outer(i, carry):
            for b in range(2):
                j = 2 * i + b
                ghs = []
                for g in range(_GPB):
                    row = j * _GPB + g
                    ghs.append(pltpu.async_copy(
                        t1_hbm.at[idx_v.at[row]],
                        rows1_v.at[b].at[pl.ds(g * _GRP, _GRP)], gsem))
                    ghs.append(pltpu.async_copy(
                        t2_hbm.at[idx_v.at[row]],
                        rows2_v.at[b].at[pl.ds(g * _GRP, _GRP)], gsem))

                pb = 1 - b
                pbase = base + (j - 1) * _BLK

                @pl.when(j > 0)
                def _fire_writes():
                    pltpu.async_copy(
                        rows1_v.at[pb],
                        out_hbm.at[pl.ds(pbase, _BLK), pl.ds(0, _HALF)],
                        wsem)
                    pltpu.async_copy(
                        rows2_v.at[pb],
                        out_hbm.at[pl.ds(pbase, _BLK), pl.ds(_HALF, _HALF)],
                        wsem)

                for h in ghs:
                    h.wait()

                @pl.when(j > 0)
                def _wait_writes():
                    pltpu.make_async_copy(
                        rows1_v.at[pb],
                        out_hbm.at[pl.ds(pbase, _BLK), pl.ds(0, _HALF)],
                        wsem).wait()
                    pltpu.make_async_copy(
                        rows2_v.at[pb],
                        out_hbm.at[pl.ds(pbase, _BLK), pl.ds(_HALF, _HALF)],
                        wsem).wait()
            return carry

        lax.fori_loop(0, _NBLK // 2, outer, 0)

        lbase = base + (_NBLK - 1) * _BLK
        pltpu.sync_copy(
            rows1_v.at[1], out_hbm.at[pl.ds(lbase, _BLK), pl.ds(0, _HALF)])
        pltpu.sync_copy(
            rows2_v.at[1],
            out_hbm.at[pl.ds(lbase, _BLK), pl.ds(_HALF, _HALF)])

    return k(x_grp, table1, table2)


def kernel(x, table1, table2):
    t1t = jnp.swapaxes(table1, 0, 1)
    t2t = jnp.swapaxes(table2, 0, 1)
    tail1 = lax.slice(table1, (_VT_FULL * _GRP, 0),
                      (_NUM_EMB, _HALF)).reshape(_TAIL // 4, 128)
    tail2 = lax.slice(table2, (_VT_FULL * _GRP, 0),
                      (_NUM_EMB, _HALF)).reshape(_TAIL // 4, 128)
    r1, r2 = _relayout(t1t, t2t, tail1, tail2)
    t1_lin = r1.reshape(_NUM_EMB, _HALF)
    t2_lin = r2.reshape(_NUM_EMB, _HALF)
    x_grp = x.reshape(_NW, _NG, _GRP).astype(jnp.int32)
    out = _dual_gather(x_grp, t1_lin, t2_lin)
    return out.reshape(_BATCH, _FIELDS, 2 * _HALF)


# speedup vs baseline: 1.8099x; 1.8099x over previous
import functools

import jax
import jax.numpy as jnp
from jax import lax
from jax.experimental import pallas as pl
from jax.experimental.pallas import tpu as pltpu
from jax.experimental.pallas import tpu_sc as plsc

_NUM_EMB = 1000000
_HALF = 32
_BATCH = 16384
_FIELDS = 26
_BF = _BATCH * _FIELDS
_NC = 2
_NS = 16
_NW = _NC * _NS
_PER_W = _BF // _NW
_GRP = 128
_NG = _PER_W // _GRP
_BLK = 512
_GPB = _BLK // _GRP
_NBLK = _PER_W // _BLK

_VT_FULL = _NUM_EMB // _GRP
_VT_ROUNDS = -(-_VT_FULL // _NW)
_TAIL = _NUM_EMB - _VT_FULL * _GRP
_OROWS = _NUM_EMB * _HALF // 128


_RB = 8192
_RGRID = -(-_NUM_EMB // _RB)


def _relayout_tc(t1t, t2t):
    def body(in1_ref, in2_ref, o1_ref, o2_ref):
        o1_ref[...] = in1_ref[...].T
        o2_ref[...] = in2_ref[...].T

    otype = jax.ShapeDtypeStruct((_NUM_EMB, _HALF), jnp.float32)
    ispec = pl.BlockSpec((_HALF, _RB), lambda i: (0, i))
    ospec = pl.BlockSpec((_RB, _HALF), lambda i: (i, 0))
    return pl.pallas_call(
        body,
        grid=(_RGRID,),
        in_specs=[ispec, ispec],
        out_specs=[ospec, ospec],
        out_shape=(otype, otype),
    )(t1t, t2t)


def _dual_gather(x_grp, table1, table2):
    mesh = plsc.VectorSubcoreMesh(core_axis_name="c", subcore_axis_name="s")

    @functools.partial(
        pl.kernel,
        mesh=mesh,
        compiler_params=pltpu.CompilerParams(use_tc_tiling_on_sc=False),
        out_type=jax.ShapeDtypeStruct((_BF, 2 * _HALF), jnp.float32),
        scratch_types=[
            pltpu.VMEM((_NG, _GRP), jnp.int32),
            pltpu.VMEM((2, _BLK, _HALF), jnp.float32),
            pltpu.VMEM((2, _BLK, _HALF), jnp.float32),
            pltpu.SemaphoreType.DMA,
            pltpu.SemaphoreType.DMA,
        ],
    )
    def k(x_hbm, t1_hbm, t2_hbm, out_hbm, idx_v, rows1_v, rows2_v,
          gsem, wsem):
        wid = lax.axis_index("s") * _NC + lax.axis_index("c")
        base = wid * _PER_W
        pltpu.sync_copy(x_hbm.at[wid], idx_v)

        def outer(i, carry):
            for b in range(2):
                j = 2 * i + b
                ghs = []
                for g in range(_GPB):
                    row = j * _GPB + g
                    ghs.append(pltpu.async_copy(
                        t1_hbm.at[idx_v.at[row]],
                        rows1_v.at[b].at[pl.ds(g * _GRP, _GRP)], gsem))
                    ghs.append(pltpu.async_copy(
                        t2_hbm.at[idx_v.at[row]],
                        rows2_v.at[b].at[pl.ds(g * _GRP, _GRP)], gsem))

                pb = 1 - b
                pbase = base + (j - 1) * _BLK

                @pl.when(j > 0)
                def _fire_writes():
                    pltpu.async_copy(
                        rows1_v.at[pb],
                        out_hbm.at[pl.ds(pbase, _BLK), pl.ds(0, _HALF)],
                        wsem)
                    pltpu.async_copy(
                        rows2_v.at[pb],
                        out_hbm.at[pl.ds(pbase, _BLK), pl.ds(_HALF, _HALF)],
                        wsem)

                for h in ghs:
                    h.wait()

                @pl.when(j > 0)
                def _wait_writes():
                    pltpu.make_async_copy(
                        rows1_v.at[pb],
                        out_hbm.at[pl.ds(pbase, _BLK), pl.ds(0, _HALF)],
                        wsem).wait()
                    pltpu.make_async_copy(
                        rows2_v.at[pb],
                        out_hbm.at[pl.ds(pbase, _BLK), pl.ds(_HALF, _HALF)],
                        wsem).wait()
            return carry

        lax.fori_loop(0, _NBLK // 2, outer, 0)

        lbase = base + (_NBLK - 1) * _BLK
        pltpu.sync_copy(
            rows1_v.at[1], out_hbm.at[pl.ds(lbase, _BLK), pl.ds(0, _HALF)])
        pltpu.sync_copy(
            rows2_v.at[1],
            out_hbm.at[pl.ds(lbase, _BLK), pl.ds(_HALF, _HALF)])

    return k(x_grp, table1, table2)


def kernel(x, table1, table2):
    t1t = jnp.swapaxes(table1, 0, 1)
    t2t = jnp.swapaxes(table2, 0, 1)
    r1, r2 = _relayout_tc(t1t, t2t)
    x_grp = x.reshape(_NW, _NG, _GRP).astype(jnp.int32)
    out = _dual_gather(x_grp, r1, r2)
    return out.reshape(_BATCH, _FIELDS, 2 * _HALF)


# speedup vs baseline: 2.3323x; 1.2886x over previous
import functools

import jax
import jax.numpy as jnp
from jax import lax
from jax.experimental import pallas as pl
from jax.experimental.pallas import tpu as pltpu
from jax.experimental.pallas import tpu_sc as plsc

_NUM_EMB = 1000000
_HALF = 32
_BATCH = 16384
_FIELDS = 26
_BF = _BATCH * _FIELDS
_NC = 2
_NS = 16
_NW = _NC * _NS
_PER_W = _BF // _NW
_GRP = 128
_NG = _PER_W // _GRP
_BLK = 512
_GPB = _BLK // _GRP
_NBLK = _PER_W // _BLK

_VT_FULL = _NUM_EMB // _GRP
_VT_ROUNDS = -(-_VT_FULL // _NW)
_TAIL = _NUM_EMB - _VT_FULL * _GRP
_OROWS = _NUM_EMB * _HALF // 128


_RB = 8192
_RGRID = -(-_NUM_EMB // _RB)


def _relayout_tc(t1t, t2t):
    def body(in1_ref, in2_ref, o1_ref, o2_ref):
        for in_ref, o_ref in ((in1_ref, o1_ref), (in2_ref, o2_ref)):
            t = in_ref[...].T.reshape(_RB // 4, 4, _HALF)
            o_ref[...] = jnp.concatenate(
                [t[:, v, :] for v in range(4)], axis=1)

    otype = jax.ShapeDtypeStruct((_OROWS, 128), jnp.float32)
    ispec = pl.BlockSpec((_HALF, _RB), lambda i: (0, i))
    ospec = pl.BlockSpec((_RB // 4, 128), lambda i: (i, 0))
    return pl.pallas_call(
        body,
        grid=(_RGRID,),
        in_specs=[ispec, ispec],
        out_specs=[ospec, ospec],
        out_shape=(otype, otype),
    )(t1t, t2t)


def _dual_gather(x_grp, table1, table2):
    mesh = plsc.VectorSubcoreMesh(core_axis_name="c", subcore_axis_name="s")

    @functools.partial(
        pl.kernel,
        mesh=mesh,
        compiler_params=pltpu.CompilerParams(use_tc_tiling_on_sc=False),
        out_type=jax.ShapeDtypeStruct((_BF, 2 * _HALF), jnp.float32),
        scratch_types=[
            pltpu.VMEM((_NG, _GRP), jnp.int32),
            pltpu.VMEM((2, _BLK, _HALF), jnp.float32),
            pltpu.VMEM((2, _BLK, _HALF), jnp.float32),
            pltpu.SemaphoreType.DMA,
            pltpu.SemaphoreType.DMA,
        ],
    )
    def k(x_hbm, t1_hbm, t2_hbm, out_hbm, idx_v, rows1_v, rows2_v,
          gsem, wsem):
        wid = lax.axis_index("s") * _NC + lax.axis_index("c")
        base = wid * _PER_W
        pltpu.sync_copy(x_hbm.at[wid], idx_v)

        def outer(i, carry):
            for b in range(2):
                j = 2 * i + b
                ghs = []
                for g in range(_GPB):
                    row = j * _GPB + g
                    ghs.append(pltpu.async_copy(
                        t1_hbm.at[idx_v.at[row]],
                        rows1_v.at[b].at[pl.ds(g * _GRP, _GRP)], gsem))
                    ghs.append(pltpu.async_copy(
                        t2_hbm.at[idx_v.at[row]],
                        rows2_v.at[b].at[pl.ds(g * _GRP, _GRP)], gsem))

                pb = 1 - b
                pbase = base + (j - 1) * _BLK

                @pl.when(j > 0)
                def _fire_writes():
                    pltpu.async_copy(
                        rows1_v.at[pb],
                        out_hbm.at[pl.ds(pbase, _BLK), pl.ds(0, _HALF)],
                        wsem)
                    pltpu.async_copy(
                        rows2_v.at[pb],
                        out_hbm.at[pl.ds(pbase, _BLK), pl.ds(_HALF, _HALF)],
                        wsem)

                for h in ghs:
                    h.wait()

                @pl.when(j > 0)
                def _wait_writes():
                    pltpu.make_async_copy(
                        rows1_v.at[pb],
                        out_hbm.at[pl.ds(pbase, _BLK), pl.ds(0, _HALF)],
                        wsem).wait()
                    pltpu.make_async_copy(
                        rows2_v.at[pb],
                        out_hbm.at[pl.ds(pbase, _BLK), pl.ds(_HALF, _HALF)],
                        wsem).wait()
            return carry

        lax.fori_loop(0, _NBLK // 2, outer, 0)

        lbase = base + (_NBLK - 1) * _BLK
        pltpu.sync_copy(
            rows1_v.at[1], out_hbm.at[pl.ds(lbase, _BLK), pl.ds(0, _HALF)])
        pltpu.sync_copy(
            rows2_v.at[1],
            out_hbm.at[pl.ds(lbase, _BLK), pl.ds(_HALF, _HALF)])

    return k(x_grp, table1, table2)


def kernel(x, table1, table2):
    t1t = jnp.swapaxes(table1, 0, 1)
    t2t = jnp.swapaxes(table2, 0, 1)
    r1, r2 = _relayout_tc(t1t, t2t)
    x_grp = x.reshape(_NW, _NG, _GRP).astype(jnp.int32)
    out = _dual_gather(x_grp, r1.reshape(_NUM_EMB, _HALF),
                       r2.reshape(_NUM_EMB, _HALF))
    return out.reshape(_BATCH, _FIELDS, 2 * _HALF)
